# Initial kernel scaffold; baseline (speedup 1.0000x reference)
import jax
import jax.numpy as jnp
from jax.experimental import pallas as pl


def kernel(X):
    raise NotImplementedError("write your pallas kernel here")


# speedup vs baseline: 15.3315x; 15.3315x over previous
import jax
import jax.numpy as jnp
from jax import lax
from jax.experimental import pallas as pl
from jax.experimental.pallas import tpu as pltpu
from jax.experimental.pallas import tpu_sc as plsc

L = 16
K = 32
B, S, H = 64, 8192, 64
CAP = 144
NV = CAP // L
SLAB = 512
NSLAB = S // SLAB
NG = H // L
NWORK = 32
BPW = B // NWORK

NEG = float("-inf")


def _iota():
    return lax.iota(jnp.int32, L)


def _rev(x):
    return lax.rev(x, (0,))


def _merge_pair_desc(a, ai, b, bi):
    rb, rbi = _rev(b), _rev(bi)
    ge = a >= rb
    p = jnp.where(ge, a, rb)
    pi = jnp.where(ge, ai, rbi)
    q = jnp.where(ge, rb, a)
    qi = jnp.where(ge, rbi, ai)
    p, pi = plsc.sort_key_val(p, pi, descending=True)
    q, qi = plsc.sort_key_val(q, qi, descending=True)
    return p, q, pi, qi


def _combine_desc(A, Bb):
    A0, A1, Ai0, Ai1 = A
    B0, B1, Bi0, Bi1 = Bb
    rB0, rBi0 = _rev(B0), _rev(Bi0)
    rB1, rBi1 = _rev(B1), _rev(Bi1)
    g0 = A0 >= rB1
    h0 = jnp.where(g0, A0, rB1)
    h0i = jnp.where(g0, Ai0, rBi1)
    g1 = A1 >= rB0
    h1 = jnp.where(g1, A1, rB0)
    h1i = jnp.where(g1, Ai1, rBi0)
    g = h0 >= h1
    u = jnp.where(g, h0, h1)
    ui = jnp.where(g, h0i, h1i)
    w = jnp.where(g, h1, h0)
    wi = jnp.where(g, h1i, h0i)
    u, ui = plsc.sort_key_val(u, ui, descending=True)
    w, wi = plsc.sort_key_val(w, wi, descending=True)
    return u, w, ui, wi


def _top32(bv, bix, c, n):
    iota = _iota()
    neg = jnp.full((L,), NEG, jnp.float32)
    leaves = []
    for j in range(NV):
        valid = (j * L + iota) < n
        kv = jnp.where(valid, bv[c, pl.ds(j * L, L)], neg)
        ki = bix[c, pl.ds(j * L, L)]
        kv, ki = plsc.sort_key_val(kv, ki, descending=True)
        leaves.append((kv, ki))
    if len(leaves) % 2:
        leaves.append((neg, jnp.zeros((L,), jnp.int32)))
    blocks = []
    for j in range(0, len(leaves), 2):
        a, ai = leaves[j]
        b2, bi2 = leaves[j + 1]
        blocks.append(_merge_pair_desc(a, ai, b2, bi2))
    while len(blocks) > 1:
        nxt = []
        for j in range(0, len(blocks) - 1, 2):
            nxt.append(_combine_desc(blocks[j], blocks[j + 1]))
        if len(blocks) % 2:
            nxt.append(blocks[-1])
        blocks = nxt
    return blocks[0]


def _lane_of(vec, c, init):
    iota = _iota()
    return lax.reduce_max(jnp.where(iota == c, vec, init), (0,))


def _reselect(bv, bix, cnt_ref, thr_ref):
    iota = _iota()

    def body(c, _):
        cntv = cnt_ref[...]
        n = _lane_of(cntv, c, jnp.int32(0))
        v0, v1, i0, i1 = _top32(bv, bix, c, n)
        bv[c, pl.ds(0, L)] = v0
        bv[c, pl.ds(L, L)] = v1
        bix[c, pl.ds(0, L)] = i0
        bix[c, pl.ds(L, L)] = i1
        ncl = jnp.minimum(n, K)
        cnt_ref[...] = jnp.where(iota == c, ncl, cntv)
        t32 = lax.reduce_min(v1, (0,))
        thr = jnp.where(n >= K, t32, jnp.float32(NEG))
        thr_ref[...] = jnp.where(iota == c, thr, thr_ref[...])
        return 0

    lax.fori_loop(0, L, body, 0)


def _scan_group(g, slab_ref, bv, bix, cnt_ref, thr_ref, s_base):
    iota = _iota()
    one = jnp.ones((L,), jnp.int32)
    zero = jnp.zeros((L,), jnp.int32)

    def block(bi, _):
        thr = thr_ref[...]
        cnt = cnt_ref[...]
        base = bi * L
        for r in range(L):
            v = slab_ref[base + r, pl.ds(g * L, L)]
            m = v > thr
            s_vec = zero + (s_base + base + r)
            plsc.store_scatter(bv, [iota, cnt], v, mask=m)
            plsc.store_scatter(bix, [iota, cnt], s_vec, mask=m)
            cnt = cnt + jnp.where(m, one, zero)
        cnt_ref[...] = cnt
        mx = lax.reduce_max(cnt, (0,))

        @pl.when(mx >= CAP - L)
        def _():
            _reselect(bv, bix, cnt_ref, thr_ref)

        return 0

    lax.fori_loop(0, SLAB // L, block, 0)


def _final_group(g, bv, bix, cnt_ref, out_tile):
    iota = _iota()

    def body(c, _):
        n = _lane_of(cnt_ref[...], c, jnp.int32(0))
        v0, v1, i0, i1 = _top32(bv, bix, c, n)
        k0, w0 = plsc.sort_key_val(i0, v0)
        k1, w1 = plsc.sort_key_val(i1, v1)
        rk1, rw1 = _rev(k1), _rev(w1)
        le = k0 <= rk1
        lo = jnp.where(le, k0, rk1)
        loV = jnp.where(le, w0, rw1)
        hi = jnp.where(le, rk1, k0)
        hiV = jnp.where(le, rw1, w0)
        _, u0 = plsc.sort_key_val(lo, loV)
        _, u1 = plsc.sort_key_val(hi, hiV)
        h_vec = jnp.zeros((L,), jnp.int32) + (g * L + c)
        plsc.store_scatter(out_tile, [iota, h_vec], u0)
        plsc.store_scatter(out_tile, [iota + L, h_vec], u1)
        return 0

    lax.fori_loop(0, L, body, 0)


def _sc_body(x_hbm, out_hbm, slab_ref,
             bv0, bv1, bv2, bv3, bx0, bx1, bx2, bx3,
             cn0, cn1, cn2, cn3, th0, th1, th2, th3, out_tile):
    bvs = (bv0, bv1, bv2, bv3)
    bxs = (bx0, bx1, bx2, bx3)
    cns = (cn0, cn1, cn2, cn3)
    ths = (th0, th1, th2, th3)
    cid = lax.axis_index("c")
    sid = lax.axis_index("s")
    wid = sid * 2 + cid

    for bb in range(BPW):
        b = wid * BPW + bb
        for g in range(NG):
            cns[g][...] = jnp.zeros((L,), jnp.int32)
            ths[g][...] = jnp.full((L,), NEG, jnp.float32)

        def slab_loop(si, _):
            pltpu.sync_copy(x_hbm.at[b, pl.ds(si * SLAB, SLAB), :], slab_ref)
            for g in range(NG):
                _scan_group(g, slab_ref, bvs[g], bxs[g], cns[g], ths[g],
                            si * SLAB)
            return 0

        lax.fori_loop(0, NSLAB, slab_loop, 0)

        for g in range(NG):
            _final_group(g, bvs[g], bxs[g], cns[g], out_tile)
        pltpu.sync_copy(out_tile, out_hbm.at[b])


def kernel(X):
    mesh = plsc.VectorSubcoreMesh(core_axis_name="c", subcore_axis_name="s")
    scratch = (
        [pltpu.VMEM((SLAB, H), jnp.float32)]
        + [pltpu.VMEM((L, CAP), jnp.float32) for _ in range(NG)]
        + [pltpu.VMEM((L, CAP), jnp.int32) for _ in range(NG)]
        + [pltpu.VMEM((L,), jnp.int32) for _ in range(NG)]
        + [pltpu.VMEM((L,), jnp.float32) for _ in range(NG)]
        + [pltpu.VMEM((K, H), jnp.float32)]
    )
    f = pl.kernel(
        _sc_body,
        out_type=jax.ShapeDtypeStruct((B, K, H), jnp.float32),
        mesh=mesh,
        scratch_types=scratch,
        compiler_params=pltpu.CompilerParams(
            use_tc_tiling_on_sc=False, needs_layout_passes=False),
    )
    return f(X)
